# Initial kernel scaffold; baseline (speedup 1.0000x reference)
import jax
import jax.numpy as jnp
from jax.experimental import pallas as pl


def kernel(point_idx_a, coord_a, a_F, point_idx_b, coord_b, b_F, W_fuse, b_fuse):
    raise NotImplementedError("write your pallas kernel here")


# speedup vs baseline: 88.2137x; 88.2137x over previous
import functools

import jax
import jax.numpy as jnp
from jax import lax
from jax.experimental import pallas as pl
from jax.experimental.pallas import tpu as pltpu
from jax.experimental.pallas import tpu_sc as plsc

N_A = 16384
N_B = 16384
D = 256
TOPK = 3
FULL_SCALE = 128
R = 0.5

BLK_A = 256
BLK_B = 2048
N_BLK_B = N_B // BLK_B
GRID_A = N_A // BLK_A
BIG = 1 << 30

SC_WORKERS = 32
SC_TOTAL = N_A * TOPK
SC_PER_W = SC_TOTAL // SC_WORKERS
SC_CHUNK = 96
SC_N_CHUNK = SC_PER_W // SC_CHUNK


def _mm_body(aF_ref, bF_ref, W_ref, bfuse_ref, A_ref, G_ref):
    W1 = W_ref[0:D, :]
    W2 = W_ref[D:2 * D, :]
    A_ref[...] = (jnp.dot(aF_ref[...], W2, preferred_element_type=jnp.float32)
                  + bfuse_ref[...])
    G_ref[...] = jnp.dot(bF_ref[...], W1 - W2,
                         preferred_element_type=jnp.float32)


def _precompute_ag(a_F, b_F, W_fuse, b_fuse):
    return pl.pallas_call(
        _mm_body,
        grid=(GRID_A,),
        in_specs=[
            pl.BlockSpec((BLK_A, D), lambda i: (i, 0)),
            pl.BlockSpec((BLK_A, D), lambda i: (i, 0)),
            pl.BlockSpec((2 * D, D), lambda i: (0, 0)),
            pl.BlockSpec((1, D), lambda i: (0, 0)),
        ],
        out_specs=[
            pl.BlockSpec((BLK_A, D), lambda i: (i, 0)),
            pl.BlockSpec((BLK_A, D), lambda i: (i, 0)),
        ],
        out_shape=[
            jax.ShapeDtypeStruct((N_A, D), jnp.float32),
            jax.ShapeDtypeStruct((N_B, D), jnp.float32),
        ],
    )(a_F, b_F, W_fuse, b_fuse.reshape(1, D))


def _knn_body(qa_ref, ga_ref, qb_ref, gb_ref, idx_ref, w_ref):
    qa = qa_ref[...]
    ga = jnp.min(ga_ref[...], axis=1, keepdims=True)
    an = jnp.sum(qa * qa, axis=1, keepdims=True)
    iota = lax.broadcasted_iota(jnp.int32, (BLK_A, BLK_B), 1)
    big = jnp.int32(BIG)

    r0 = jnp.full((BLK_A, 1), big, jnp.int32)
    r1 = r0
    r2 = r0
    for t in range(N_BLK_B):
        qb = qb_ref[:, t * BLK_B:(t + 1) * BLK_B]
        gb = jnp.min(gb_ref[:, t * BLK_B:(t + 1) * BLK_B], axis=0,
                     keepdims=True)
        xy = jnp.dot(qa, qb, preferred_element_type=jnp.float32)
        bn = jnp.sum(qb * qb, axis=0, keepdims=True)
        d2 = (an + bn) - 2.0 * xy
        d2i = d2.astype(jnp.int32)
        keys = jnp.where(ga == gb, d2i * 16384 + (iota + t * BLK_B), big)
        for _ in range(TOPK):
            m = jnp.min(keys, axis=1, keepdims=True)
            keys = jnp.where(keys == m, big, keys)
            h0 = jnp.maximum(r0, m)
            r0 = jnp.minimum(r0, m)
            h1 = jnp.maximum(r1, h0)
            r1 = jnp.minimum(r1, h0)
            r2 = jnp.minimum(r2, h1)

    rr = jnp.concatenate([r0, r1, r2], axis=1)
    d2f = (rr >> 14).astype(jnp.float32)
    dist = jnp.sqrt(d2f) * (1.0 / FULL_SCALE)
    idx_ref[...] = rr & 16383
    w_ref[...] = jnp.maximum(0.0, R - dist)


def _knn(qa, ga_pad, qb, gb_pad):
    return pl.pallas_call(
        _knn_body,
        grid=(GRID_A,),
        in_specs=[
            pl.BlockSpec((BLK_A, 8), lambda i: (i, 0)),
            pl.BlockSpec((BLK_A, 8), lambda i: (i, 0)),
            pl.BlockSpec((8, N_B), lambda i: (0, 0)),
            pl.BlockSpec((8, N_B), lambda i: (0, 0)),
        ],
        out_specs=[
            pl.BlockSpec((BLK_A, TOPK), lambda i: (i, 0)),
            pl.BlockSpec((BLK_A, TOPK), lambda i: (i, 0)),
        ],
        out_shape=[
            jax.ShapeDtypeStruct((N_A, TOPK), jnp.int32),
            jax.ShapeDtypeStruct((N_A, TOPK), jnp.float32),
        ],
    )(qa, ga_pad, qb, gb_pad)


@functools.cache
def _build_sc_gather():
    mesh = plsc.VectorSubcoreMesh(core_axis_name="c", subcore_axis_name="s")

    @functools.partial(
        pl.kernel,
        mesh=mesh,
        out_type=jax.ShapeDtypeStruct((SC_TOTAL, D), jnp.float32),
        scratch_types=[
            pltpu.VMEM((SC_CHUNK,), jnp.int32),
            pltpu.VMEM((SC_CHUNK, D), jnp.float32),
            pltpu.SemaphoreType.DMA,
        ],
    )
    def sc_gather(table_hbm, idx_hbm, out_hbm, idx_v, rows_v, sem):
        wid = lax.axis_index("s") * 2 + lax.axis_index("c")
        base = wid * SC_PER_W
        for c in range(SC_N_CHUNK):
            off = base + c * SC_CHUNK
            pltpu.sync_copy(idx_hbm.at[pl.ds(off, SC_CHUNK)], idx_v)
            pltpu.async_copy(table_hbm.at[idx_v], rows_v, sem).wait()
            pltpu.sync_copy(rows_v, out_hbm.at[pl.ds(off, SC_CHUNK)])

    return sc_gather


def _combine_body(aF_ref, A_ref, rows_ref, w_ref, out_ref):
    a3 = A_ref[...][:, None, :]
    r3 = rows_ref[...]
    w3 = w_ref[...][:, :, None]
    tmp = jnp.sum(jnp.maximum(a3 + r3, 0.0) * w3, axis=1)
    out_ref[:, 0:D] = aF_ref[...]
    out_ref[:, D:2 * D] = tmp


def _combine(a_F, A, rows3, w):
    return pl.pallas_call(
        _combine_body,
        grid=(GRID_A,),
        in_specs=[
            pl.BlockSpec((BLK_A, D), lambda i: (i, 0)),
            pl.BlockSpec((BLK_A, D), lambda i: (i, 0)),
            pl.BlockSpec((BLK_A, TOPK, D), lambda i: (i, 0, 0)),
            pl.BlockSpec((BLK_A, TOPK), lambda i: (i, 0)),
        ],
        out_specs=pl.BlockSpec((BLK_A, 2 * D), lambda i: (i, 0)),
        out_shape=jax.ShapeDtypeStruct((N_A, 2 * D), jnp.float32),
    )(a_F, A, rows3, w)


def kernel(point_idx_a, coord_a, a_F, point_idx_b, coord_b, b_F,
           W_fuse, b_fuse):
    ca = coord_a.astype(jnp.float32)
    cb = coord_b.astype(jnp.float32)
    qa = jnp.pad(ca, ((0, 0), (0, 5)))
    qb = jnp.pad(cb, ((0, 0), (0, 5))).T
    ga_pad = jnp.broadcast_to(point_idx_a[:, None].astype(jnp.int32),
                              (N_A, 8))
    gb_pad = jnp.broadcast_to(point_idx_b[None, :].astype(jnp.int32),
                              (8, N_B))

    A, G = _precompute_ag(a_F, b_F, W_fuse, b_fuse)
    idx, w = _knn(qa, ga_pad, qb, gb_pad)
    rows = _build_sc_gather()(G, idx.reshape(-1))
    return _combine(a_F, A, rows.reshape(N_A, TOPK, D), w)


# speedup vs baseline: 146.6596x; 1.6625x over previous
import functools

import jax
import jax.numpy as jnp
from jax import lax
from jax.experimental import pallas as pl
from jax.experimental.pallas import tpu as pltpu
from jax.experimental.pallas import tpu_sc as plsc

N_A = 16384
N_B = 16384
D = 256
TOPK = 3
FULL_SCALE = 128
R = 0.5

BLK_A = 256
BLK_B = 2048
N_BLK_B = N_B // BLK_B
GRID_A = N_A // BLK_A
BIG = 1 << 30

SC_WORKERS = 32
SC_TOTAL = N_A * TOPK
SC_PER_W = SC_TOTAL // SC_WORKERS
SC_CHUNK = 96
SC_N_CHUNK = SC_PER_W // SC_CHUNK


def _mm_body(aF_ref, bF_ref, W_ref, bfuse_ref, A_ref, G_ref):
    W1 = W_ref[0:D, :]
    W2 = W_ref[D:2 * D, :]
    A_ref[...] = (jnp.dot(aF_ref[...], W2, preferred_element_type=jnp.float32)
                  + bfuse_ref[...])
    G_ref[...] = jnp.dot(bF_ref[...], W1 - W2,
                         preferred_element_type=jnp.float32)


def _precompute_ag(a_F, b_F, W_fuse, b_fuse):
    return pl.pallas_call(
        _mm_body,
        grid=(GRID_A,),
        in_specs=[
            pl.BlockSpec((BLK_A, D), lambda i: (i, 0)),
            pl.BlockSpec((BLK_A, D), lambda i: (i, 0)),
            pl.BlockSpec((2 * D, D), lambda i: (0, 0)),
            pl.BlockSpec((1, D), lambda i: (0, 0)),
        ],
        out_specs=[
            pl.BlockSpec((BLK_A, D), lambda i: (i, 0)),
            pl.BlockSpec((BLK_A, D), lambda i: (i, 0)),
        ],
        out_shape=[
            jax.ShapeDtypeStruct((N_A, D), jnp.float32),
            jax.ShapeDtypeStruct((N_B, D), jnp.float32),
        ],
    )(a_F, b_F, W_fuse, b_fuse.reshape(1, D))


def _knn_body(qa_ref, ga_ref, qb_ref, gb_ref, idx_ref, w_ref):
    qa = qa_ref[...]
    ga = jnp.min(ga_ref[...], axis=1, keepdims=True)
    an = jnp.sum(qa * qa, axis=1, keepdims=True)
    iota = lax.broadcasted_iota(jnp.int32, (BLK_A, BLK_B), 1)
    big = jnp.int32(BIG)

    gb_row = jnp.min(gb_ref[...], axis=0, keepdims=True)
    g_lo = jnp.min(ga)
    g_hi = jnp.max(ga)
    cnt_lo = jnp.sum((gb_row < g_lo).astype(jnp.int32))
    cnt_hi = jnp.sum((gb_row <= g_hi).astype(jnp.int32))
    t_lo = cnt_lo // BLK_B
    t_hi = (cnt_hi + BLK_B - 1) // BLK_B

    r0 = jnp.full((BLK_A, 1), big, jnp.int32)

    def scan_block(t, carry):
        r0, r1, r2 = carry
        base = t * BLK_B
        qb = qb_ref[:, pl.ds(base, BLK_B)]
        gb = jnp.min(gb_ref[:, pl.ds(base, BLK_B)], axis=0,
                     keepdims=True)
        xy = jnp.dot(qa, qb, preferred_element_type=jnp.float32)
        bn = jnp.sum(qb * qb, axis=0, keepdims=True)
        d2 = (an + bn) - 2.0 * xy
        d2i = d2.astype(jnp.int32)
        keys = jnp.where(ga == gb, d2i * 16384 + (iota + base), big)
        for _ in range(TOPK):
            m = jnp.min(keys, axis=1, keepdims=True)
            keys = jnp.where(keys == m, big, keys)
            h0 = jnp.maximum(r0, m)
            r0 = jnp.minimum(r0, m)
            h1 = jnp.maximum(r1, h0)
            r1 = jnp.minimum(r1, h0)
            r2 = jnp.minimum(r2, h1)
        return r0, r1, r2

    r0, r1, r2 = lax.fori_loop(t_lo, t_hi, scan_block, (r0, r0, r0))

    rr = jnp.concatenate([r0, r1, r2], axis=1)
    d2f = (rr >> 14).astype(jnp.float32)
    dist = jnp.sqrt(d2f) * (1.0 / FULL_SCALE)
    idx_ref[...] = rr & 16383
    w_ref[...] = jnp.maximum(0.0, R - dist)


def _knn(qa, ga_pad, qb, gb_pad):
    return pl.pallas_call(
        _knn_body,
        grid=(GRID_A,),
        in_specs=[
            pl.BlockSpec((BLK_A, 8), lambda i: (i, 0)),
            pl.BlockSpec((BLK_A, 8), lambda i: (i, 0)),
            pl.BlockSpec((8, N_B), lambda i: (0, 0)),
            pl.BlockSpec((8, N_B), lambda i: (0, 0)),
        ],
        out_specs=[
            pl.BlockSpec((BLK_A, TOPK), lambda i: (i, 0)),
            pl.BlockSpec((BLK_A, TOPK), lambda i: (i, 0)),
        ],
        out_shape=[
            jax.ShapeDtypeStruct((N_A, TOPK), jnp.int32),
            jax.ShapeDtypeStruct((N_A, TOPK), jnp.float32),
        ],
    )(qa, ga_pad, qb, gb_pad)


@functools.cache
def _build_sc_gather():
    mesh = plsc.VectorSubcoreMesh(core_axis_name="c", subcore_axis_name="s")

    @functools.partial(
        pl.kernel,
        mesh=mesh,
        out_type=jax.ShapeDtypeStruct((SC_TOTAL, D), jnp.float32),
        scratch_types=[
            pltpu.VMEM((SC_CHUNK,), jnp.int32),
            pltpu.VMEM((SC_CHUNK, D), jnp.float32),
            pltpu.SemaphoreType.DMA,
        ],
    )
    def sc_gather(table_hbm, idx_hbm, out_hbm, idx_v, rows_v, sem):
        wid = lax.axis_index("s") * 2 + lax.axis_index("c")
        base = wid * SC_PER_W
        for c in range(SC_N_CHUNK):
            off = base + c * SC_CHUNK
            pltpu.sync_copy(idx_hbm.at[pl.ds(off, SC_CHUNK)], idx_v)
            pltpu.async_copy(table_hbm.at[idx_v], rows_v, sem).wait()
            pltpu.sync_copy(rows_v, out_hbm.at[pl.ds(off, SC_CHUNK)])

    return sc_gather


def _combine_body(aF_ref, A_ref, rows_ref, w_ref, out_ref):
    a3 = A_ref[...][:, None, :]
    r3 = rows_ref[...]
    w3 = w_ref[...][:, :, None]
    tmp = jnp.sum(jnp.maximum(a3 + r3, 0.0) * w3, axis=1)
    out_ref[:, 0:D] = aF_ref[...]
    out_ref[:, D:2 * D] = tmp


def _combine(a_F, A, rows3, w):
    return pl.pallas_call(
        _combine_body,
        grid=(GRID_A,),
        in_specs=[
            pl.BlockSpec((BLK_A, D), lambda i: (i, 0)),
            pl.BlockSpec((BLK_A, D), lambda i: (i, 0)),
            pl.BlockSpec((BLK_A, TOPK, D), lambda i: (i, 0, 0)),
            pl.BlockSpec((BLK_A, TOPK), lambda i: (i, 0)),
        ],
        out_specs=pl.BlockSpec((BLK_A, 2 * D), lambda i: (i, 0)),
        out_shape=jax.ShapeDtypeStruct((N_A, 2 * D), jnp.float32),
    )(a_F, A, rows3, w)


def kernel(point_idx_a, coord_a, a_F, point_idx_b, coord_b, b_F,
           W_fuse, b_fuse):
    ca = coord_a.astype(jnp.float32)
    cb = coord_b.astype(jnp.float32)
    qa = jnp.pad(ca, ((0, 0), (0, 5)))
    qb = jnp.pad(cb, ((0, 0), (0, 5))).T
    ga_pad = jnp.broadcast_to(point_idx_a[:, None].astype(jnp.int32),
                              (N_A, 8))
    gb_pad = jnp.broadcast_to(point_idx_b[None, :].astype(jnp.int32),
                              (8, N_B))

    A, G = _precompute_ag(a_F, b_F, W_fuse, b_fuse)
    idx, w = _knn(qa, ga_pad, qb, gb_pad)
    rows = _build_sc_gather()(G, idx.reshape(-1))
    return _combine(a_F, A, rows.reshape(N_A, TOPK, D), w)


# speedup vs baseline: 149.6202x; 1.0202x over previous
import functools

import jax
import jax.numpy as jnp
from jax import lax
from jax.experimental import pallas as pl
from jax.experimental.pallas import tpu as pltpu
from jax.experimental.pallas import tpu_sc as plsc

N_A = 16384
N_B = 16384
D = 256
TOPK = 3
FULL_SCALE = 128
R = 0.5

BLK_A = 256
BLK_B = 2048
N_BLK_B = N_B // BLK_B
GRID_A = N_A // BLK_A
BIG = 1 << 30

SC_WORKERS = 32
SC_TOTAL = N_A * TOPK
SC_PER_W = SC_TOTAL // SC_WORKERS
SC_CHUNK = 96
SC_N_CHUNK = SC_PER_W // SC_CHUNK


def _mm_body(aF_ref, bF_ref, W_ref, bfuse_ref, A_ref, G_ref):
    W1 = W_ref[0:D, :]
    W2 = W_ref[D:2 * D, :]
    A_ref[...] = (jnp.dot(aF_ref[...], W2, preferred_element_type=jnp.float32)
                  + bfuse_ref[...])
    G_ref[...] = jnp.dot(bF_ref[...], W1 - W2,
                         preferred_element_type=jnp.float32)


def _precompute_ag(a_F, b_F, W_fuse, b_fuse):
    return pl.pallas_call(
        _mm_body,
        grid=(GRID_A,),
        in_specs=[
            pl.BlockSpec((BLK_A, D), lambda i: (i, 0)),
            pl.BlockSpec((BLK_A, D), lambda i: (i, 0)),
            pl.BlockSpec((2 * D, D), lambda i: (0, 0)),
            pl.BlockSpec((1, D), lambda i: (0, 0)),
        ],
        out_specs=[
            pl.BlockSpec((BLK_A, D), lambda i: (i, 0)),
            pl.BlockSpec((BLK_A, D), lambda i: (i, 0)),
        ],
        out_shape=[
            jax.ShapeDtypeStruct((N_A, D), jnp.float32),
            jax.ShapeDtypeStruct((N_B, D), jnp.float32),
        ],
    )(a_F, b_F, W_fuse, b_fuse.reshape(1, D))


def _knn_body(qa_ref, ga_ref, qb_ref, gb_ref, idx_ref, w_ref):
    qa = qa_ref[...]
    ga = jnp.min(ga_ref[...], axis=1, keepdims=True)
    an = jnp.sum(qa * qa, axis=1, keepdims=True)
    iota = lax.broadcasted_iota(jnp.int32, (BLK_A, BLK_B), 1)
    big = jnp.int32(BIG)

    gb_row = jnp.min(gb_ref[...], axis=0, keepdims=True)
    g_lo = jnp.min(ga)
    g_hi = jnp.max(ga)
    cnt_lo = jnp.sum((gb_row < g_lo).astype(jnp.int32))
    cnt_hi = jnp.sum((gb_row <= g_hi).astype(jnp.int32))
    t_lo = cnt_lo // BLK_B
    t_hi = (cnt_hi + BLK_B - 1) // BLK_B

    r0 = jnp.full((BLK_A, 1), big, jnp.int32)

    def scan_block(t, carry):
        r0, r1, r2 = carry
        base = t * BLK_B
        qb = qb_ref[:, pl.ds(base, BLK_B)]
        xy = jnp.dot(qa, qb, preferred_element_type=jnp.float32)
        bn = jnp.sum(qb * qb, axis=0, keepdims=True)
        d2 = (an + bn) - 2.0 * xy
        d2i = jnp.minimum(d2, 65536.0).astype(jnp.int32)
        keys = d2i * 16384 + (iota + base)
        for _ in range(TOPK):
            m = jnp.min(keys, axis=1, keepdims=True)
            keys = jnp.where(keys == m, big, keys)
            h0 = jnp.maximum(r0, m)
            r0 = jnp.minimum(r0, m)
            h1 = jnp.maximum(r1, h0)
            r1 = jnp.minimum(r1, h0)
            r2 = jnp.minimum(r2, h1)
        return r0, r1, r2

    r0, r1, r2 = lax.fori_loop(t_lo, t_hi, scan_block, (r0, r0, r0))

    rr = jnp.concatenate([r0, r1, r2], axis=1)
    d2f = (rr >> 14).astype(jnp.float32)
    dist = jnp.sqrt(d2f) * (1.0 / FULL_SCALE)
    idx_ref[...] = rr & 16383
    w_ref[...] = jnp.maximum(0.0, R - dist)


def _knn(qa, ga_pad, qb, gb_pad):
    return pl.pallas_call(
        _knn_body,
        grid=(GRID_A,),
        in_specs=[
            pl.BlockSpec((BLK_A, 8), lambda i: (i, 0)),
            pl.BlockSpec((BLK_A, 8), lambda i: (i, 0)),
            pl.BlockSpec((8, N_B), lambda i: (0, 0)),
            pl.BlockSpec((8, N_B), lambda i: (0, 0)),
        ],
        out_specs=[
            pl.BlockSpec((BLK_A, TOPK), lambda i: (i, 0)),
            pl.BlockSpec((BLK_A, TOPK), lambda i: (i, 0)),
        ],
        out_shape=[
            jax.ShapeDtypeStruct((N_A, TOPK), jnp.int32),
            jax.ShapeDtypeStruct((N_A, TOPK), jnp.float32),
        ],
    )(qa, ga_pad, qb, gb_pad)


@functools.cache
def _build_sc_gather():
    mesh = plsc.VectorSubcoreMesh(core_axis_name="c", subcore_axis_name="s")

    @functools.partial(
        pl.kernel,
        mesh=mesh,
        out_type=jax.ShapeDtypeStruct((SC_TOTAL, D), jnp.float32),
        scratch_types=[
            pltpu.VMEM((SC_CHUNK,), jnp.int32),
            pltpu.VMEM((SC_CHUNK, D), jnp.float32),
            pltpu.SemaphoreType.DMA,
        ],
    )
    def sc_gather(table_hbm, idx_hbm, out_hbm, idx_v, rows_v, sem):
        wid = lax.axis_index("s") * 2 + lax.axis_index("c")
        base = wid * SC_PER_W
        for c in range(SC_N_CHUNK):
            off = base + c * SC_CHUNK
            pltpu.sync_copy(idx_hbm.at[pl.ds(off, SC_CHUNK)], idx_v)
            pltpu.async_copy(table_hbm.at[idx_v], rows_v, sem).wait()
            pltpu.sync_copy(rows_v, out_hbm.at[pl.ds(off, SC_CHUNK)])

    return sc_gather


def _combine_body(aF_ref, A_ref, rows_ref, w_ref, out_ref):
    a3 = A_ref[...][:, None, :]
    r3 = rows_ref[...]
    w3 = w_ref[...][:, :, None]
    tmp = jnp.sum(jnp.maximum(a3 + r3, 0.0) * w3, axis=1)
    out_ref[:, 0:D] = aF_ref[...]
    out_ref[:, D:2 * D] = tmp


def _combine(a_F, A, rows3, w):
    return pl.pallas_call(
        _combine_body,
        grid=(GRID_A,),
        in_specs=[
            pl.BlockSpec((BLK_A, D), lambda i: (i, 0)),
            pl.BlockSpec((BLK_A, D), lambda i: (i, 0)),
            pl.BlockSpec((BLK_A, TOPK, D), lambda i: (i, 0, 0)),
            pl.BlockSpec((BLK_A, TOPK), lambda i: (i, 0)),
        ],
        out_specs=pl.BlockSpec((BLK_A, 2 * D), lambda i: (i, 0)),
        out_shape=jax.ShapeDtypeStruct((N_A, 2 * D), jnp.float32),
    )(a_F, A, rows3, w)


def _prep(point_idx_a, coord_a, point_idx_b, coord_b):
    ca = coord_a.astype(jnp.float32)
    cb = coord_b.astype(jnp.float32)
    pa = (point_idx_a.astype(jnp.float32) * 221.0)[:, None]
    pb = (point_idx_b.astype(jnp.float32) * 221.0)[:, None]
    qa = jnp.pad(jnp.concatenate([ca, pa], axis=1), ((0, 0), (0, 4)))
    qb = jnp.pad(jnp.concatenate([cb, pb], axis=1), ((0, 0), (0, 4))).T
    ga_pad = jnp.broadcast_to(point_idx_a[:, None].astype(jnp.int32),
                              (N_A, 8))
    gb_pad = jnp.broadcast_to(point_idx_b[None, :].astype(jnp.int32),
                              (8, N_B))
    return qa, ga_pad, qb, gb_pad


def kernel(point_idx_a, coord_a, a_F, point_idx_b, coord_b, b_F,
           W_fuse, b_fuse):
    qa, ga_pad, qb, gb_pad = _prep(point_idx_a, coord_a,
                                   point_idx_b, coord_b)

    A, G = _precompute_ag(a_F, b_F, W_fuse, b_fuse)
    idx, w = _knn(qa, ga_pad, qb, gb_pad)
    rows = _build_sc_gather()(G, idx.reshape(-1))
    return _combine(a_F, A, rows.reshape(N_A, TOPK, D), w)
